# baseline (speedup 1.0000x reference)
import dataclasses
import functools

import jax
import jax.numpy as jnp
import numpy as np
from jax import lax
from jax.experimental import pallas as pl
from jax.experimental.pallas import tpu as pltpu
from jax.experimental.pallas import tpu_sc as plsc

DIM = 768
K_DIM = 512
N_KEYS = 512
HEADS = 4
KNN = 32
HALF = K_DIM // 2

_CANDS = [(i, j) for i in range(KNN) for j in range(KNN) if (i + 1) * (j + 1) <= KNN]
_NCAND = 128
_P1 = np.zeros((KNN, _NCAND), np.float32)
_P2 = np.zeros((KNN, _NCAND), np.float32)
for _p, (_i, _j) in enumerate(_CANDS):
    _P1[_i, _p] = 1.0
    _P2[_j, _p] = 1.0
_CBIAS = np.zeros((1, _NCAND), np.float32)
_CBIAS[0, len(_CANDS):] = -1e30


def _split3(s):
    p0 = s.astype(jnp.bfloat16)
    r = s - p0.astype(jnp.float32)
    p1 = r.astype(jnp.bfloat16)
    p2 = (r - p1.astype(jnp.float32)).astype(jnp.bfloat16)
    return p0, p1, p2


def _exact_sel(parts, P):
    acc = None
    for p in parts:
        d = jax.lax.dot_general(p, P, (((1,), (0,)), ((), ())),
                                preferred_element_type=jnp.float32)
        acc = d if acc is None else acc + d
    return acc


def _topk_exact(s, k, payload=None):
    bt, n = s.shape
    lane = lax.broadcasted_iota(jnp.int32, (bt, n), 1).astype(jnp.float32)
    outlane = lax.broadcasted_iota(jnp.int32, (bt, k), 1)
    neg = jnp.float32(-jnp.inf)

    def extract(s, t, ov, oi):
        m = jnp.max(s, axis=1, keepdims=True)
        first = jnp.min(jnp.where(s == m, lane, jnp.float32(n)), axis=1,
                        keepdims=True)
        sel = lane == first
        if payload is not None:
            pv = jnp.sum(jnp.where(sel, payload, jnp.float32(0.0)), axis=1,
                         keepdims=True)
        else:
            pv = first
        s = jnp.where(sel, neg, s)
        ov = jnp.where(outlane == t, m, ov)
        oi = jnp.where(outlane == t, pv, oi)
        return s, ov, oi

    unroll = 8

    def body(t, carry):
        s, ov, oi = carry
        for u in range(unroll):
            s, ov, oi = extract(s, unroll * t + u, ov, oi)
        return s, ov, oi

    _, vals, idxs = lax.fori_loop(
        0, k // unroll, body,
        (s, jnp.zeros((bt, k), jnp.float32), jnp.zeros((bt, k), jnp.float32)))
    return vals, idxs


def _tc_body(x_ref, wq_ref, bq_ref, keys_ref, p1_ref, p2_ref, cb_ref,
             idx_ref, w_ref):
    x = x_ref[...]
    bt = x.shape[0]
    q = jnp.dot(x, wq_ref[...], preferred_element_type=jnp.float32)
    q = q + bq_ref[...]
    p1 = p1_ref[...]
    p2 = p2_ref[...]
    cbias = cb_ref[...]
    ss = []
    for h in range(HEADS):
        for c in range(2):
            qhc = q[:, h * K_DIM + c * HALF: h * K_DIM + (c + 1) * HALF]
            ss.append(jax.lax.dot_general(qhc, keys_ref[h, c],
                                          (((1,), (1,)), ((), ())),
                                          preferred_element_type=jnp.float32))
    s_all = jnp.concatenate(ss, axis=0)
    tv, ti = _topk_exact(s_all, KNN)
    s1 = jnp.concatenate([tv[2 * h * bt:(2 * h + 1) * bt] for h in range(HEADS)], axis=0)
    s2 = jnp.concatenate([tv[(2 * h + 1) * bt:(2 * h + 2) * bt] for h in range(HEADS)], axis=0)
    i1f = jnp.concatenate([ti[2 * h * bt:(2 * h + 1) * bt] for h in range(HEADS)], axis=0)
    i2f = jnp.concatenate([ti[(2 * h + 1) * bt:(2 * h + 2) * bt] for h in range(HEADS)], axis=0)
    cand_s = (_exact_sel(_split3(s1), p1) + _exact_sel(_split3(s2), p2)
              + cbias)
    hi1 = jnp.floor(i1f * (1.0 / 32.0))
    lo1 = i1f - hi1 * 32.0
    hi2 = jnp.floor(i2f * (1.0 / 32.0))
    lo2 = i2f - hi2 * 32.0
    c_i1 = (_exact_sel([hi1.astype(jnp.bfloat16)], p1) * 32.0
            + _exact_sel([lo1.astype(jnp.bfloat16)], p1))
    c_i2 = (_exact_sel([hi2.astype(jnp.bfloat16)], p2) * 32.0
            + _exact_sel([lo2.astype(jnp.bfloat16)], p2))
    cand_idx = c_i1 * jnp.float32(N_KEYS) + c_i2
    cs, cif = _topk_exact(cand_s, KNN, payload=cand_idx)
    cif = jnp.minimum(cif, jnp.float32(N_KEYS * N_KEYS - 1))
    sc_all = jnp.concatenate([cs[h * bt:(h + 1) * bt] for h in range(HEADS)], axis=1)
    idx_all = jnp.concatenate([cif[h * bt:(h + 1) * bt] for h in range(HEADS)], axis=1)
    m = jnp.max(sc_all, axis=1, keepdims=True)
    e = jnp.exp(sc_all - m)
    wgt = e / jnp.sum(e, axis=1, keepdims=True)
    idx_ref[...] = idx_all.astype(jnp.int32)
    w_ref[...] = wgt


def _tc_scores_topk(xf, Wq, bq2, keys):
    b = xf.shape[0]
    bt = 256
    grid = (b // bt,)
    return pl.pallas_call(
        _tc_body,
        grid=grid,
        in_specs=[
            pl.BlockSpec((bt, DIM), lambda i: (i, 0)),
            pl.BlockSpec((DIM, HEADS * K_DIM), lambda i: (0, 0)),
            pl.BlockSpec((1, HEADS * K_DIM), lambda i: (0, 0)),
            pl.BlockSpec((HEADS, 2, N_KEYS, HALF), lambda i: (0, 0, 0, 0)),
            pl.BlockSpec((KNN, _NCAND), lambda i: (0, 0)),
            pl.BlockSpec((KNN, _NCAND), lambda i: (0, 0)),
            pl.BlockSpec((1, _NCAND), lambda i: (0, 0)),
        ],
        out_specs=[
            pl.BlockSpec((bt, HEADS * KNN), lambda i: (i, 0)),
            pl.BlockSpec((bt, HEADS * KNN), lambda i: (i, 0)),
        ],
        out_shape=[
            jax.ShapeDtypeStruct((b, HEADS * KNN), jnp.int32),
            jax.ShapeDtypeStruct((b, HEADS * KNN), jnp.float32),
        ],
    )(xf, Wq, bq2, keys,
      jnp.asarray(_P1, jnp.bfloat16), jnp.asarray(_P2, jnp.bfloat16),
      jnp.asarray(_CBIAS))


_NC = 2
_NS = 16
_NW = _NC * _NS
_CHUNK = 32
_GRP = 16


def _sc_body(values_hbm, idx_hbm, w_hbm, out_hbm,
             idx_v, w_v, rows_v, out_v, sem_r0, sem_r1, sem_o):
    k = HEADS * KNN
    tpw = idx_hbm.shape[0] // _NW
    grp = min(_GRP, tpw)
    nchunk = k // _CHUNK
    wid = lax.axis_index("s") * _NC + lax.axis_index("c")
    base = wid * tpw
    pltpu.sync_copy(idx_hbm.at[pl.ds(base, tpw)], idx_v)
    pltpu.sync_copy(w_hbm.at[pl.ds(base, tpw)], w_v)
    sem_rows = (sem_r0, sem_r1)

    def issue(t, c, slot):
        pltpu.async_copy(
            values_hbm.at[idx_v.at[t, pl.ds(c * _CHUNK, _CHUNK)]],
            rows_v.at[slot], sem_rows[slot])

    issue(0, 0, 0)

    half = DIM // 2
    ngrp = tpw // grp

    def grp_body(g, _):
        oslot = g % 2

        def token_body(tl, _):
            t = g * grp + tl
            for c in range(nchunk):
                slot = c % 2
                if c + 1 < nchunk:
                    issue(t, c + 1, (c + 1) % 2)
                else:
                    @pl.when(t + 1 < tpw)
                    def _():
                        issue(t + 1, 0, 0)
                pltpu.make_async_copy(
                    values_hbm.at[idx_v.at[t, pl.ds(c * _CHUNK, _CHUNK)]],
                    rows_v.at[slot], sem_rows[slot]).wait()

                for p in range(2):
                    def row_body(r2, acc):
                        r = 2 * r2
                        wv0 = plsc.load_gather(
                            w_v, [jnp.full((16,), t, jnp.int32),
                                  jnp.full((16,), c * _CHUNK + r, jnp.int32)])
                        wv1 = plsc.load_gather(
                            w_v, [jnp.full((16,), t, jnp.int32),
                                  jnp.full((16,), c * _CHUNK + r + 1,
                                           jnp.int32)])
                        return tuple(
                            (acc[j]
                             + rows_v[slot, r, pl.ds(p * half + j * 16, 16)]
                             * wv0)
                            + rows_v[slot, r + 1, pl.ds(p * half + j * 16, 16)]
                            * wv1
                            for j in range(half // 16))

                    acc = lax.fori_loop(0, _CHUNK // 2, row_body,
                                        (jnp.zeros((16,), jnp.float32),)
                                        * (half // 16))
                    for j in range(half // 16):
                        off = p * half + j * 16
                        if c == 0:
                            out_v[oslot, tl, pl.ds(off, 16)] = acc[j]
                        else:
                            out_v[oslot, tl, pl.ds(off, 16)] += acc[j]
            return 0

        lax.fori_loop(0, grp, token_body, 0)
        @pl.when(g > 0)
        def _():
            pltpu.make_async_copy(
                out_v.at[1 - oslot],
                out_hbm.at[pl.ds(base + (g - 1) * grp, grp)],
                sem_o).wait()
        pltpu.async_copy(out_v.at[oslot],
                         out_hbm.at[pl.ds(base + g * grp, grp)],
                         sem_o)
        return 0

    lax.fori_loop(0, ngrp, grp_body, 0)
    pltpu.make_async_copy(
        out_v.at[(ngrp - 1) % 2],
        out_hbm.at[pl.ds(base + (ngrp - 1) * grp, grp)],
        sem_o).wait()


def _sc_weighted_gather(values, idx, w):
    b = idx.shape[0]
    k = HEADS * KNN
    tpw = b // _NW
    mesh = plsc.VectorSubcoreMesh(core_axis_name="c", subcore_axis_name="s")
    cp = pltpu.CompilerParams()
    if "needs_layout_passes" in pltpu.CompilerParams.__dataclass_fields__:
        cp = dataclasses.replace(cp, needs_layout_passes=False)
    kern = pl.kernel(
        _sc_body,
        out_type=jax.ShapeDtypeStruct((b, DIM), jnp.float32),
        mesh=mesh,
        scratch_types=[
            pltpu.VMEM((tpw, k), jnp.int32),
            pltpu.VMEM((tpw, k), jnp.float32),
            pltpu.VMEM((2, _CHUNK, DIM), jnp.float32),
            pltpu.VMEM((2, min(_GRP, tpw), DIM), jnp.float32),
            pltpu.SemaphoreType.DMA,
            pltpu.SemaphoreType.DMA,
            pltpu.SemaphoreType.DMA,
        ],
        compiler_params=cp,
    )
    return kern(values, idx, w)


def kernel(x, Wq, bq, keys, values):
    prefix = x.shape[:-1]
    xf = x.reshape(-1, DIM)
    b = xf.shape[0]
    nslice = 16
    bs = b // nslice
    bq2 = bq.reshape(1, -1)
    outs = []
    for i in range(nslice):
        idx, w = _tc_scores_topk(xf[i * bs:(i + 1) * bs], Wq, bq2, keys)
        outs.append(_sc_weighted_gather(values, idx, w))
    out = jnp.concatenate(outs, axis=0)
    return out.reshape(prefix + (DIM,))


# speedup vs baseline: 1.0215x; 1.0215x over previous
import dataclasses
import functools

import jax
import jax.numpy as jnp
import numpy as np
from jax import lax
from jax.experimental import pallas as pl
from jax.experimental.pallas import tpu as pltpu
from jax.experimental.pallas import tpu_sc as plsc

DIM = 768
K_DIM = 512
N_KEYS = 512
HEADS = 4
KNN = 32
HALF = K_DIM // 2

_CANDS = [(i, j) for i in range(KNN) for j in range(KNN) if (i + 1) * (j + 1) <= KNN]
_NCAND = 128
_P1 = np.zeros((KNN, _NCAND), np.float32)
_P2 = np.zeros((KNN, _NCAND), np.float32)
for _p, (_i, _j) in enumerate(_CANDS):
    _P1[_i, _p] = 1.0
    _P2[_j, _p] = 1.0
_CBIAS = np.zeros((1, _NCAND), np.float32)
_CBIAS[0, len(_CANDS):] = -1e30


def _split3(s):
    p0 = s.astype(jnp.bfloat16)
    r = s - p0.astype(jnp.float32)
    p1 = r.astype(jnp.bfloat16)
    p2 = (r - p1.astype(jnp.float32)).astype(jnp.bfloat16)
    return p0, p1, p2


def _exact_sel(parts, P):
    acc = None
    for p in parts:
        d = jax.lax.dot_general(p, P, (((1,), (0,)), ((), ())),
                                preferred_element_type=jnp.float32)
        acc = d if acc is None else acc + d
    return acc


def _topk_exact(s, k, payload=None):
    bt, n = s.shape
    lane = lax.broadcasted_iota(jnp.int32, (bt, n), 1).astype(jnp.float32)
    outlane = lax.broadcasted_iota(jnp.int32, (bt, k), 1)
    neg = jnp.float32(-jnp.inf)

    def extract(s, t, ov, oi):
        m = jnp.max(s, axis=1, keepdims=True)
        first = jnp.min(jnp.where(s == m, lane, jnp.float32(n)), axis=1,
                        keepdims=True)
        sel = lane == first
        if payload is not None:
            pv = jnp.sum(jnp.where(sel, payload, jnp.float32(0.0)), axis=1,
                         keepdims=True)
        else:
            pv = first
        s = jnp.where(sel, neg, s)
        ov = jnp.where(outlane == t, m, ov)
        oi = jnp.where(outlane == t, pv, oi)
        return s, ov, oi

    unroll = 8

    def body(t, carry):
        s, ov, oi = carry
        for u in range(unroll):
            s, ov, oi = extract(s, unroll * t + u, ov, oi)
        return s, ov, oi

    _, vals, idxs = lax.fori_loop(
        0, k // unroll, body,
        (s, jnp.zeros((bt, k), jnp.float32), jnp.zeros((bt, k), jnp.float32)))
    return vals, idxs


def _tc_body(x_ref, wq_ref, bq_ref, keys_ref, p1_ref, p2_ref, cb_ref,
             idx_ref, w_ref):
    x = x_ref[...]
    bt = x.shape[0]
    q = jnp.dot(x, wq_ref[...], preferred_element_type=jnp.float32)
    q = q + bq_ref[...]
    p1 = p1_ref[...]
    p2 = p2_ref[...]
    cbias = cb_ref[...]
    ss = []
    for h in range(HEADS):
        for c in range(2):
            qhc = q[:, h * K_DIM + c * HALF: h * K_DIM + (c + 1) * HALF]
            ss.append(jax.lax.dot_general(qhc, keys_ref[h, c],
                                          (((1,), (1,)), ((), ())),
                                          preferred_element_type=jnp.float32))
    s_all = jnp.concatenate(ss, axis=0)
    tv, ti = _topk_exact(s_all, KNN)
    s1 = jnp.concatenate([tv[2 * h * bt:(2 * h + 1) * bt] for h in range(HEADS)], axis=0)
    s2 = jnp.concatenate([tv[(2 * h + 1) * bt:(2 * h + 2) * bt] for h in range(HEADS)], axis=0)
    i1f = jnp.concatenate([ti[2 * h * bt:(2 * h + 1) * bt] for h in range(HEADS)], axis=0)
    i2f = jnp.concatenate([ti[(2 * h + 1) * bt:(2 * h + 2) * bt] for h in range(HEADS)], axis=0)
    cand_s = (_exact_sel(_split3(s1), p1) + _exact_sel(_split3(s2), p2)
              + cbias)
    hi1 = jnp.floor(i1f * (1.0 / 32.0))
    lo1 = i1f - hi1 * 32.0
    hi2 = jnp.floor(i2f * (1.0 / 32.0))
    lo2 = i2f - hi2 * 32.0
    c_i1 = (_exact_sel([hi1.astype(jnp.bfloat16)], p1) * 32.0
            + _exact_sel([lo1.astype(jnp.bfloat16)], p1))
    c_i2 = (_exact_sel([hi2.astype(jnp.bfloat16)], p2) * 32.0
            + _exact_sel([lo2.astype(jnp.bfloat16)], p2))
    cand_idx = c_i1 * jnp.float32(N_KEYS) + c_i2
    cs, cif = _topk_exact(cand_s, KNN, payload=cand_idx)
    cif = jnp.minimum(cif, jnp.float32(N_KEYS * N_KEYS - 1))
    sc_all = jnp.concatenate([cs[h * bt:(h + 1) * bt] for h in range(HEADS)], axis=1)
    idx_all = jnp.concatenate([cif[h * bt:(h + 1) * bt] for h in range(HEADS)], axis=1)
    m = jnp.max(sc_all, axis=1, keepdims=True)
    e = jnp.exp(sc_all - m)
    wgt = e / jnp.sum(e, axis=1, keepdims=True)
    idx_ref[...] = idx_all.astype(jnp.int32)
    w_ref[...] = wgt


def _tc_scores_topk(xf, Wq, bq2, keys):
    b = xf.shape[0]
    bt = min(512, b)
    grid = (b // bt,)
    return pl.pallas_call(
        _tc_body,
        grid=grid,
        in_specs=[
            pl.BlockSpec((bt, DIM), lambda i: (i, 0)),
            pl.BlockSpec((DIM, HEADS * K_DIM), lambda i: (0, 0)),
            pl.BlockSpec((1, HEADS * K_DIM), lambda i: (0, 0)),
            pl.BlockSpec((HEADS, 2, N_KEYS, HALF), lambda i: (0, 0, 0, 0)),
            pl.BlockSpec((KNN, _NCAND), lambda i: (0, 0)),
            pl.BlockSpec((KNN, _NCAND), lambda i: (0, 0)),
            pl.BlockSpec((1, _NCAND), lambda i: (0, 0)),
        ],
        out_specs=[
            pl.BlockSpec((bt, HEADS * KNN), lambda i: (i, 0)),
            pl.BlockSpec((bt, HEADS * KNN), lambda i: (i, 0)),
        ],
        out_shape=[
            jax.ShapeDtypeStruct((b, HEADS * KNN), jnp.int32),
            jax.ShapeDtypeStruct((b, HEADS * KNN), jnp.float32),
        ],
    )(xf, Wq, bq2, keys,
      jnp.asarray(_P1, jnp.bfloat16), jnp.asarray(_P2, jnp.bfloat16),
      jnp.asarray(_CBIAS))


_NC = 2
_NS = 16
_NW = _NC * _NS
_CHUNK = 32
_GRP = 16


def _sc_body(values_hbm, idx_hbm, w_hbm, out_hbm,
             idx_v, w_v, rows_v, out_v, sem_r0, sem_r1, sem_o):
    k = HEADS * KNN
    tpw = idx_hbm.shape[0] // _NW
    grp = min(_GRP, tpw)
    nchunk = k // _CHUNK
    wid = lax.axis_index("s") * _NC + lax.axis_index("c")
    base = wid * tpw
    pltpu.sync_copy(idx_hbm.at[pl.ds(base, tpw)], idx_v)
    pltpu.sync_copy(w_hbm.at[pl.ds(base, tpw)], w_v)
    sem_rows = (sem_r0, sem_r1)

    def issue(t, c, slot):
        pltpu.async_copy(
            values_hbm.at[idx_v.at[t, pl.ds(c * _CHUNK, _CHUNK)]],
            rows_v.at[slot], sem_rows[slot])

    issue(0, 0, 0)

    half = DIM // 2
    ngrp = tpw // grp

    def grp_body(g, _):
        oslot = g % 2

        def token_body(tl, _):
            t = g * grp + tl
            for c in range(nchunk):
                slot = c % 2
                if c + 1 < nchunk:
                    issue(t, c + 1, (c + 1) % 2)
                else:
                    @pl.when(t + 1 < tpw)
                    def _():
                        issue(t + 1, 0, 0)
                pltpu.make_async_copy(
                    values_hbm.at[idx_v.at[t, pl.ds(c * _CHUNK, _CHUNK)]],
                    rows_v.at[slot], sem_rows[slot]).wait()

                for p in range(2):
                    def row_body(r2, acc):
                        r = 2 * r2
                        wv0 = plsc.load_gather(
                            w_v, [jnp.full((16,), t, jnp.int32),
                                  jnp.full((16,), c * _CHUNK + r, jnp.int32)])
                        wv1 = plsc.load_gather(
                            w_v, [jnp.full((16,), t, jnp.int32),
                                  jnp.full((16,), c * _CHUNK + r + 1,
                                           jnp.int32)])
                        return tuple(
                            (acc[j]
                             + rows_v[slot, r, pl.ds(p * half + j * 16, 16)]
                             * wv0)
                            + rows_v[slot, r + 1, pl.ds(p * half + j * 16, 16)]
                            * wv1
                            for j in range(half // 16))

                    acc = lax.fori_loop(0, _CHUNK // 2, row_body,
                                        (jnp.zeros((16,), jnp.float32),)
                                        * (half // 16))
                    for j in range(half // 16):
                        off = p * half + j * 16
                        if c == 0:
                            out_v[oslot, tl, pl.ds(off, 16)] = acc[j]
                        else:
                            out_v[oslot, tl, pl.ds(off, 16)] += acc[j]
            return 0

        lax.fori_loop(0, grp, token_body, 0)
        @pl.when(g > 0)
        def _():
            pltpu.make_async_copy(
                out_v.at[1 - oslot],
                out_hbm.at[pl.ds(base + (g - 1) * grp, grp)],
                sem_o).wait()
        pltpu.async_copy(out_v.at[oslot],
                         out_hbm.at[pl.ds(base + g * grp, grp)],
                         sem_o)
        return 0

    lax.fori_loop(0, ngrp, grp_body, 0)
    pltpu.make_async_copy(
        out_v.at[(ngrp - 1) % 2],
        out_hbm.at[pl.ds(base + (ngrp - 1) * grp, grp)],
        sem_o).wait()


def _sc_weighted_gather(values, idx, w):
    b = idx.shape[0]
    k = HEADS * KNN
    tpw = b // _NW
    mesh = plsc.VectorSubcoreMesh(core_axis_name="c", subcore_axis_name="s")
    cp = pltpu.CompilerParams()
    if "needs_layout_passes" in pltpu.CompilerParams.__dataclass_fields__:
        cp = dataclasses.replace(cp, needs_layout_passes=False)
    kern = pl.kernel(
        _sc_body,
        out_type=jax.ShapeDtypeStruct((b, DIM), jnp.float32),
        mesh=mesh,
        scratch_types=[
            pltpu.VMEM((tpw, k), jnp.int32),
            pltpu.VMEM((tpw, k), jnp.float32),
            pltpu.VMEM((2, _CHUNK, DIM), jnp.float32),
            pltpu.VMEM((2, min(_GRP, tpw), DIM), jnp.float32),
            pltpu.SemaphoreType.DMA,
            pltpu.SemaphoreType.DMA,
            pltpu.SemaphoreType.DMA,
        ],
        compiler_params=cp,
    )
    return kern(values, idx, w)


def kernel(x, Wq, bq, keys, values):
    prefix = x.shape[:-1]
    xf = x.reshape(-1, DIM)
    b = xf.shape[0]
    nslice = 8
    bs = b // nslice
    bq2 = bq.reshape(1, -1)
    outs = []
    for i in range(nslice):
        idx, w = _tc_scores_topk(xf[i * bs:(i + 1) * bs], Wq, bq2, keys)
        outs.append(_sc_weighted_gather(values, idx, w))
    out = jnp.concatenate(outs, axis=0)
    return out.reshape(prefix + (DIM,))
